# baseline (speedup 1.0000x reference)
import dataclasses
import functools

import jax
import jax.numpy as jnp
from jax import lax
from jax.experimental import pallas as pl
from jax.experimental.pallas import tpu as pltpu
from jax.experimental.pallas import tpu_sc as plsc

LANES = 128
TB = 256
T = 256
NEG = -1000000000.0


def _noise_padded(b, s, e):
    noise = jax.random.normal(jax.random.key(42), (b, s, e),
                              dtype=jnp.float32) * 0.01
    return jnp.pad(noise.reshape(b * s, e), ((0, 0), (0, LANES - e)),
                   constant_values=NEG)


def _routing_body(n_tokens, n_experts, x_ref, gw_ref, gb_ref, noise_ref,
                  i1_ref, i2_ref, r1_ref, r2_ref, p1_ref, p2_ref,
                  off_ref, tstart_ref, tcnt_ref, aux_ref, carry, usage):
    blk = pl.program_id(0)
    nblk = pl.num_programs(0)

    @pl.when(blk == 0)
    def _():
        carry[...] = jnp.zeros_like(carry)
        usage[...] = jnp.zeros_like(usage)

    logits = jnp.dot(x_ref[...].astype(jnp.bfloat16),
                     gw_ref[...].astype(jnp.bfloat16),
                     preferred_element_type=jnp.float32)
    logits = logits + gb_ref[...] + noise_ref[...]

    lane = lax.broadcasted_iota(jnp.int32, (TB, LANES), 1)
    m1 = jnp.max(logits, axis=1, keepdims=True)
    i1 = jnp.min(jnp.where(logits == m1, lane, LANES), axis=1, keepdims=True)
    oh1 = lane == i1
    l2 = jnp.where(oh1, NEG, logits)
    m2 = jnp.max(l2, axis=1, keepdims=True)
    i2 = jnp.min(jnp.where(l2 == m2, lane, LANES), axis=1, keepdims=True)
    oh2 = lane == i2

    masked = jnp.where(logits >= m2, logits, NEG)
    ex = jnp.exp(masked - m1)
    probs = ex / jnp.sum(ex, axis=1, keepdims=True)
    p1 = jnp.sum(jnp.where(oh1, probs, 0.0), axis=1, keepdims=True)
    p2 = jnp.sum(jnp.where(oh2, probs, 0.0), axis=1, keepdims=True)
    usage[...] += jnp.sum(probs, axis=0, keepdims=True)

    ohb = (oh1 | oh2).astype(jnp.float32)
    row = lax.broadcasted_iota(jnp.int32, (TB, TB), 0)
    col = lax.broadcasted_iota(jnp.int32, (TB, TB), 1)
    tri = (row > col).astype(jnp.float32)
    pre = jnp.dot(tri, ohb, preferred_element_type=jnp.float32)
    base = pre + carry[...]
    r1 = jnp.sum(jnp.where(oh1, base, 0.0), axis=1, keepdims=True)
    r2 = jnp.sum(jnp.where(oh2, base, 0.0), axis=1, keepdims=True)
    carry[...] += jnp.sum(ohb, axis=0, keepdims=True)

    i1_ref[...] = i1.astype(jnp.int32)
    i2_ref[...] = i2.astype(jnp.int32)
    r1_ref[...] = r1.astype(jnp.int32)
    r2_ref[...] = r2.astype(jnp.int32)
    p1_ref[...] = p1
    p2_ref[...] = p2

    @pl.when(blk == nblk - 1)
    def _():
        counts = carry[...]
        padded = jnp.ceil(counts / T) * T
        r128 = lax.broadcasted_iota(jnp.int32, (LANES, LANES), 0)
        c128 = lax.broadcasted_iota(jnp.int32, (LANES, LANES), 1)
        ut = (r128 < c128).astype(jnp.float32)
        pad8 = jnp.broadcast_to(padded, (8, LANES))
        off_excl = jnp.dot(pad8, ut, preferred_element_type=jnp.float32)[0:1]
        off_incl = off_excl + padded
        off_ref[...] = off_excl.astype(jnp.int32)

        tstart_ref[...] = (off_excl / T).astype(jnp.int32)
        tcnt_ref[...] = (padded / T).astype(jnp.int32)

        tot = jnp.sum(usage[...])
        imp = usage[...] / tot
        mean = jnp.sum(imp) / n_experts
        lr = lax.broadcasted_iota(jnp.int32, (1, LANES), 1)
        diff = jnp.where(lr < n_experts, imp - mean, 0.0)
        var = jnp.sum(diff * diff) / n_experts
        aux = jnp.sqrt(var) / (mean + 1e-10)
        aux_ref[...] = jnp.full((1, LANES), aux, jnp.float32)


def _routing_call(x2, gw_pad, gb_pad, noise_pad, n_experts):
    n = x2.shape[0]
    d = x2.shape[1]
    nblk = n // TB
    body = functools.partial(_routing_body, n, n_experts)
    outs = (
        jax.ShapeDtypeStruct((n, 1), jnp.int32),
        jax.ShapeDtypeStruct((n, 1), jnp.int32),
        jax.ShapeDtypeStruct((n, 1), jnp.int32),
        jax.ShapeDtypeStruct((n, 1), jnp.int32),
        jax.ShapeDtypeStruct((n, 1), jnp.float32),
        jax.ShapeDtypeStruct((n, 1), jnp.float32),
        jax.ShapeDtypeStruct((1, LANES), jnp.int32),
        jax.ShapeDtypeStruct((1, LANES), jnp.int32),
        jax.ShapeDtypeStruct((1, LANES), jnp.int32),
        jax.ShapeDtypeStruct((1, LANES), jnp.float32),
    )
    tok = lambda b: (b, 0)
    fixed = lambda b: (0, 0)
    return pl.pallas_call(
        body,
        grid=(nblk,),
        in_specs=[
            pl.BlockSpec((TB, d), tok),
            pl.BlockSpec((d, LANES), fixed),
            pl.BlockSpec((1, LANES), fixed),
            pl.BlockSpec((TB, LANES), tok),
        ],
        out_specs=(
            pl.BlockSpec((TB, 1), tok), pl.BlockSpec((TB, 1), tok),
            pl.BlockSpec((TB, 1), tok), pl.BlockSpec((TB, 1), tok),
            pl.BlockSpec((TB, 1), tok), pl.BlockSpec((TB, 1), tok),
            pl.BlockSpec((1, LANES), fixed),
            pl.BlockSpec((1, LANES), fixed),
            pl.BlockSpec((1, LANES), fixed),
            pl.BlockSpec((1, LANES), fixed),
        ),
        out_shape=outs,
        scratch_shapes=[
            pltpu.VMEM((1, LANES), jnp.float32),
            pltpu.VMEM((1, LANES), jnp.float32),
        ],
        compiler_params=pltpu.CompilerParams(
            dimension_semantics=("arbitrary",)),
    )(x2, gw_pad, gb_pad, noise_pad)


def _sc_mesh():
    return plsc.VectorSubcoreMesh(core_axis_name="c", subcore_axis_name="s")


def _sc_compiler_params():
    cp = pltpu.CompilerParams()
    if "needs_layout_passes" in pltpu.CompilerParams.__dataclass_fields__:
        cp = dataclasses.replace(cp, needs_layout_passes=False)
    return cp


def _positions_call(i1, i2, r1, r2, off16, a_pad):
    n = i1.shape[0]

    @functools.partial(
        pl.kernel,
        mesh=_sc_mesh(),
        out_type=(
            jax.ShapeDtypeStruct((a_pad,), jnp.int32),
            jax.ShapeDtypeStruct((n,), jnp.int32),
            jax.ShapeDtypeStruct((n,), jnp.int32),
        ),
        scratch_types=[
            pltpu.VMEM((n,), jnp.int32), pltpu.VMEM((n,), jnp.int32),
            pltpu.VMEM((n,), jnp.int32), pltpu.VMEM((n,), jnp.int32),
            pltpu.VMEM((16,), jnp.int32),
            pltpu.VMEM((a_pad,), jnp.int32),
            pltpu.VMEM((n,), jnp.int32), pltpu.VMEM((n,), jnp.int32),
        ],
        compiler_params=_sc_compiler_params(),
    )
    def k(i1_hbm, i2_hbm, r1_hbm, r2_hbm, off_hbm,
          st_hbm, pos1_hbm, pos2_hbm,
          i1v, i2v, r1v, r2v, offv, stv, p1v, p2v):
        wid = lax.axis_index("s") * 2 + lax.axis_index("c")

        @pl.when(wid == 0)
        def _():
            pltpu.sync_copy(i1_hbm, i1v)
            pltpu.sync_copy(i2_hbm, i2v)
            pltpu.sync_copy(r1_hbm, r1v)
            pltpu.sync_copy(r2_hbm, r2v)
            pltpu.sync_copy(off_hbm, offv)
            @pl.loop(0, a_pad, step=16)
            def _(j):
                stv[pl.ds(j, 16)] = lax.rem(lax.iota(jnp.int32, 16) + j, n)

            @pl.loop(0, n, step=16)
            def _(t):
                sl = pl.ds(t, 16)
                toks = lax.iota(jnp.int32, 16) + t
                pos1 = plsc.load_gather(offv, [i1v[sl]]) + r1v[sl]
                p1v[sl] = pos1
                plsc.store_scatter(stv, [pos1], toks)
                pos2 = plsc.load_gather(offv, [i2v[sl]]) + r2v[sl]
                p2v[sl] = pos2
                plsc.store_scatter(stv, [pos2], toks)

            pltpu.sync_copy(stv, st_hbm)
            pltpu.sync_copy(p1v, pos1_hbm)
            pltpu.sync_copy(p2v, pos2_hbm)

    return k(i1, i2, r1, r2, off16)


def _gather_rows_call(table, idx, chunk, tag):
    nrows = idx.shape[0]
    d = table.shape[1]
    nw = 32
    per_w = nrows // nw
    nch = per_w // chunk

    def k(tab_hbm, idx_hbm, out_hbm, idxv, buf0, buf1,
          gs0, gs1, os0, os1):
        wid = lax.axis_index("s") * 2 + lax.axis_index("c")
        base = wid * per_w
        bufs, gsems, osems = (buf0, buf1), (gs0, gs1), (os0, os1)
        pltpu.sync_copy(idx_hbm.at[pl.ds(base, per_w)], idxv)

        def gather(c):
            return pltpu.make_async_copy(
                tab_hbm.at[idxv.at[pl.ds(c * chunk, chunk)]],
                bufs[c % 2], gsems[c % 2])

        def putout(c):
            return pltpu.make_async_copy(
                bufs[c % 2], out_hbm.at[pl.ds(base + c * chunk, chunk)],
                osems[c % 2])

        gather(0).start()
        for c in range(nch):
            if c + 1 < nch:
                if c >= 1:
                    putout(c - 1).wait()
                gather(c + 1).start()
            gather(c).wait()
            putout(c).start()
        if nch >= 2:
            putout(nch - 2).wait()
        putout(nch - 1).wait()

    k.__name__ = "gather_" + tag
    wrapped = pl.kernel(
        k,
        mesh=_sc_mesh(),
        out_type=jax.ShapeDtypeStruct((nrows, d), table.dtype),
        scratch_types=[
            pltpu.VMEM((per_w,), jnp.int32),
            pltpu.VMEM((chunk, d), table.dtype),
            pltpu.VMEM((chunk, d), table.dtype),
            pltpu.SemaphoreType.DMA, pltpu.SemaphoreType.DMA,
            pltpu.SemaphoreType.DMA, pltpu.SemaphoreType.DMA,
        ],
    )
    return wrapped(table, idx)


def _gemm_body(n_experts, tinfo_ref, xs_hbm, w1_ref, b1_ref, w2_ref,
               b2_ref, ys_hbm, xbuf, ybuf, xsem, ysem):
    e = pl.program_id(0)
    start = tinfo_ref[e]
    cnt = tinfo_ref[n_experts + e]

    def tile(j, _):
        row = (start + j) * T
        cp = pltpu.make_async_copy(xs_hbm.at[pl.ds(row, T)], xbuf, xsem)
        cp.start()
        cp.wait()
        h = jnp.dot(xbuf[...], w1_ref[0], preferred_element_type=jnp.float32)
        h = h + b1_ref[0]
        h = h * jax.nn.sigmoid(h)
        out = jnp.dot(h, w2_ref[0], preferred_element_type=jnp.float32)
        out = out + b2_ref[0]
        d2 = out.shape[1] // 2
        lo = lax.bitcast_convert_type(
            out[:, :d2].astype(jnp.bfloat16), jnp.uint16).astype(jnp.uint32)
        hi = lax.bitcast_convert_type(
            out[:, d2:].astype(jnp.bfloat16), jnp.uint16).astype(jnp.uint32)

        @pl.when(j > 0)
        def _():
            pltpu.make_async_copy(
                ybuf, ys_hbm.at[pl.ds(row - T, T)], ysem).wait()

        ybuf[...] = lax.bitcast_convert_type(lo | (hi << 16), jnp.int32)
        pltpu.make_async_copy(ybuf, ys_hbm.at[pl.ds(row, T)], ysem).start()
        return 0

    lax.fori_loop(0, cnt, tile, 0)

    @pl.when(cnt > 0)
    def _():
        pltpu.make_async_copy(
            ybuf, ys_hbm.at[pl.ds((start + cnt - 1) * T, T)], ysem).wait()


def _gemm_call(tinfo, xs, w1, b1, w2, b2):
    a_pad, d = xs.shape
    f = w1.shape[2]
    e = w1.shape[0]
    body = functools.partial(_gemm_body, e)
    grid_spec = pltpu.PrefetchScalarGridSpec(
        num_scalar_prefetch=1,
        grid=(e,),
        in_specs=[
            pl.BlockSpec(memory_space=pl.ANY),
            pl.BlockSpec((1, d, f), lambda i, tinfo: (i, 0, 0)),
            pl.BlockSpec((1, 1, f), lambda i, tinfo: (i, 0, 0)),
            pl.BlockSpec((1, f, d), lambda i, tinfo: (i, 0, 0)),
            pl.BlockSpec((1, 1, d), lambda i, tinfo: (i, 0, 0)),
        ],
        out_specs=pl.BlockSpec(memory_space=pl.ANY),
        scratch_shapes=[
            pltpu.VMEM((T, d), jnp.float32),
            pltpu.VMEM((T, d // 2), jnp.int32),
            pltpu.SemaphoreType.DMA, pltpu.SemaphoreType.DMA,
        ],
    )
    return pl.pallas_call(
        body,
        grid_spec=grid_spec,
        out_shape=jax.ShapeDtypeStruct((a_pad, d // 2), jnp.int32),
        compiler_params=pltpu.CompilerParams(
            dimension_semantics=("arbitrary",)),
    )(tinfo, xs, w1, b1, w2, b2)


def _unpack_bf16_pair(g):
    u = lax.bitcast_convert_type(g, jnp.uint32)
    lo = lax.bitcast_convert_type(
        (u & 0xFFFF).astype(jnp.uint16), jnp.bfloat16).astype(jnp.float32)
    hi = lax.bitcast_convert_type(
        (u >> 16).astype(jnp.uint16), jnp.bfloat16).astype(jnp.float32)
    return lo, hi


def _combine_body(g1_ref, g2_ref, p1_ref, p2_ref, o_ref):
    lo1, hi1 = _unpack_bf16_pair(g1_ref[...])
    lo2, hi2 = _unpack_bf16_pair(g2_ref[...])
    p1 = p1_ref[...]
    p2 = p2_ref[...]
    d2 = lo1.shape[1]
    o_ref[:, :d2] = p1 * lo1 + p2 * lo2
    o_ref[:, d2:] = p1 * hi1 + p2 * hi2


def _combine_call(g, p1, p2):
    n = p1.shape[0]
    d2 = g.shape[1]
    nblk = n // TB
    return pl.pallas_call(
        _combine_body,
        grid=(nblk,),
        in_specs=[
            pl.BlockSpec((TB, d2), lambda b: (b, 0)),
            pl.BlockSpec((TB, d2), lambda b: (b + nblk, 0)),
            pl.BlockSpec((TB, 1), lambda b: (b, 0)),
            pl.BlockSpec((TB, 1), lambda b: (b, 0)),
        ],
        out_specs=pl.BlockSpec((TB, 2 * d2), lambda b: (b, 0)),
        out_shape=jax.ShapeDtypeStruct((n, 2 * d2), jnp.float32),
        compiler_params=pltpu.CompilerParams(
            dimension_semantics=("parallel",)),
    )(g, g, p1, p2)


def kernel(x, gate_w, gate_b, w1, b1, w2, b2):
    x = jnp.asarray(x, jnp.float32)
    b, s, d = x.shape
    e = gate_w.shape[1]
    f = w1.shape[2]
    n = b * s
    a_pad = ((n * 2 + e * T) // T) * T

    x2 = x.reshape(n, d)
    gw_pad = jnp.pad(gate_w, ((0, 0), (0, LANES - e)))
    gb_pad = jnp.pad(gate_b, (0, LANES - e)).reshape(1, LANES)
    noise_pad = _noise_padded(b, s, e)

    i1, i2, r1, r2, p1, p2, off, tstart, tcnt, auxv = _routing_call(
        x2, gw_pad, gb_pad, noise_pad, e)

    off16 = off[0, :16]
    st, pos1, pos2 = _positions_call(
        i1.reshape(n), i2.reshape(n), r1.reshape(n), r2.reshape(n),
        off16, a_pad)

    xs = _gather_rows_call(x2, st, chunk=32, tag="dispatch")

    tinfo = jnp.concatenate([tstart[0, :e], tcnt[0, :e]])
    ys = _gemm_call(tinfo, xs, w1, b1.reshape(e, 1, f), w2,
                    b2.reshape(e, 1, d))

    g = _gather_rows_call(ys, jnp.concatenate([pos1, pos2]), chunk=64,
                          tag="combine")
    out = _combine_call(g, p1, p2)

    final = out.reshape(b, s, d)
    topk = jnp.concatenate([i1, i2], axis=1).reshape(b, s, 2)
    aux = auxv[0, 0]
    return (final, topk, aux)


# speedup vs baseline: 1.1955x; 1.1955x over previous
import dataclasses
import functools

import jax
import jax.numpy as jnp
from jax import lax
from jax.experimental import pallas as pl
from jax.experimental.pallas import tpu as pltpu
from jax.experimental.pallas import tpu_sc as plsc

LANES = 128
TB = 256
T = 256
NEG = -1000000000.0


def _noise_padded(b, s, e):
    noise = jax.random.normal(jax.random.key(42), (b, s, e),
                              dtype=jnp.float32) * 0.01
    return jnp.pad(noise.reshape(b * s, e), ((0, 0), (0, LANES - e)),
                   constant_values=NEG)


def _routing_body(n_tokens, n_experts, x_ref, gw_ref, gb_ref, noise_ref,
                  i1_ref, i2_ref, r1_ref, r2_ref, p1_ref, p2_ref,
                  off_ref, te_ref, aux_ref, carry, usage):
    blk = pl.program_id(0)
    nblk = pl.num_programs(0)

    @pl.when(blk == 0)
    def _():
        carry[...] = jnp.zeros_like(carry)
        usage[...] = jnp.zeros_like(usage)

    logits = jnp.dot(x_ref[...].astype(jnp.bfloat16),
                     gw_ref[...].astype(jnp.bfloat16),
                     preferred_element_type=jnp.float32)
    logits = logits + gb_ref[...] + noise_ref[...]

    lane = lax.broadcasted_iota(jnp.int32, (TB, LANES), 1)
    m1 = jnp.max(logits, axis=1, keepdims=True)
    i1 = jnp.min(jnp.where(logits == m1, lane, LANES), axis=1, keepdims=True)
    oh1 = lane == i1
    l2 = jnp.where(oh1, NEG, logits)
    m2 = jnp.max(l2, axis=1, keepdims=True)
    i2 = jnp.min(jnp.where(l2 == m2, lane, LANES), axis=1, keepdims=True)
    oh2 = lane == i2

    masked = jnp.where(logits >= m2, logits, NEG)
    ex = jnp.exp(masked - m1)
    probs = ex / jnp.sum(ex, axis=1, keepdims=True)
    p1 = jnp.sum(jnp.where(oh1, probs, 0.0), axis=1, keepdims=True)
    p2 = jnp.sum(jnp.where(oh2, probs, 0.0), axis=1, keepdims=True)
    usage[...] += jnp.sum(probs, axis=0, keepdims=True)

    ohb = (oh1 | oh2).astype(jnp.float32)
    row = lax.broadcasted_iota(jnp.int32, (TB, TB), 0)
    col = lax.broadcasted_iota(jnp.int32, (TB, TB), 1)
    tri = (row > col).astype(jnp.float32)
    pre = jnp.dot(tri, ohb, preferred_element_type=jnp.float32)
    base = pre + carry[...]
    r1 = jnp.sum(jnp.where(oh1, base, 0.0), axis=1, keepdims=True)
    r2 = jnp.sum(jnp.where(oh2, base, 0.0), axis=1, keepdims=True)
    carry[...] += jnp.sum(ohb, axis=0, keepdims=True)

    i1_ref[...] = i1.astype(jnp.int32)
    i2_ref[...] = i2.astype(jnp.int32)
    r1_ref[...] = r1.astype(jnp.int32)
    r2_ref[...] = r2.astype(jnp.int32)
    p1_ref[...] = p1
    p2_ref[...] = p2

    @pl.when(blk == nblk - 1)
    def _():
        counts = carry[...]
        padded = jnp.ceil(counts / T) * T
        r128 = lax.broadcasted_iota(jnp.int32, (LANES, LANES), 0)
        c128 = lax.broadcasted_iota(jnp.int32, (LANES, LANES), 1)
        ut = (r128 < c128).astype(jnp.float32)
        pad8 = jnp.broadcast_to(padded, (8, LANES))
        off_excl = jnp.dot(pad8, ut, preferred_element_type=jnp.float32)[0:1]
        off_incl = off_excl + padded
        off_ref[...] = off_excl.astype(jnp.int32)

        tstart = lax.broadcasted_iota(jnp.int32, (LANES, 1), 0) * T
        offi = jnp.broadcast_to(off_incl, (LANES, LANES)).astype(jnp.int32)
        m = jnp.where(c128 < n_experts, (tstart >= offi).astype(jnp.int32), 0)
        te = jnp.sum(m, axis=1, keepdims=True)
        te_ref[...] = jnp.where(te >= n_experts, n_experts - 1 + 8, te)

        tot = jnp.sum(usage[...])
        imp = usage[...] / tot
        mean = jnp.sum(imp) / n_experts
        lr = lax.broadcasted_iota(jnp.int32, (1, LANES), 1)
        diff = jnp.where(lr < n_experts, imp - mean, 0.0)
        var = jnp.sum(diff * diff) / n_experts
        aux = jnp.sqrt(var) / (mean + 1e-10)
        aux_ref[...] = jnp.full((1, LANES), aux, jnp.float32)


def _routing_call(x2, gw_pad, gb_pad, noise_pad, n_experts):
    n = x2.shape[0]
    d = x2.shape[1]
    nblk = n // TB
    body = functools.partial(_routing_body, n, n_experts)
    outs = (
        jax.ShapeDtypeStruct((n, 1), jnp.int32),
        jax.ShapeDtypeStruct((n, 1), jnp.int32),
        jax.ShapeDtypeStruct((n, 1), jnp.int32),
        jax.ShapeDtypeStruct((n, 1), jnp.int32),
        jax.ShapeDtypeStruct((n, 1), jnp.float32),
        jax.ShapeDtypeStruct((n, 1), jnp.float32),
        jax.ShapeDtypeStruct((1, LANES), jnp.int32),
        jax.ShapeDtypeStruct((LANES, 1), jnp.int32),
        jax.ShapeDtypeStruct((1, LANES), jnp.float32),
    )
    tok = lambda b: (b, 0)
    fixed = lambda b: (0, 0)
    return pl.pallas_call(
        body,
        grid=(nblk,),
        in_specs=[
            pl.BlockSpec((TB, d), tok),
            pl.BlockSpec((d, LANES), fixed),
            pl.BlockSpec((1, LANES), fixed),
            pl.BlockSpec((TB, LANES), tok),
        ],
        out_specs=(
            pl.BlockSpec((TB, 1), tok), pl.BlockSpec((TB, 1), tok),
            pl.BlockSpec((TB, 1), tok), pl.BlockSpec((TB, 1), tok),
            pl.BlockSpec((TB, 1), tok), pl.BlockSpec((TB, 1), tok),
            pl.BlockSpec((1, LANES), fixed),
            pl.BlockSpec((LANES, 1), fixed),
            pl.BlockSpec((1, LANES), fixed),
        ),
        out_shape=outs,
        scratch_shapes=[
            pltpu.VMEM((1, LANES), jnp.float32),
            pltpu.VMEM((1, LANES), jnp.float32),
        ],
        compiler_params=pltpu.CompilerParams(
            dimension_semantics=("arbitrary",)),
    )(x2, gw_pad, gb_pad, noise_pad)


def _sc_mesh():
    return plsc.VectorSubcoreMesh(core_axis_name="c", subcore_axis_name="s")


def _sc_compiler_params():
    cp = pltpu.CompilerParams()
    if "needs_layout_passes" in pltpu.CompilerParams.__dataclass_fields__:
        cp = dataclasses.replace(cp, needs_layout_passes=False)
    return cp


def _positions_call(i1, i2, r1, r2, off16, a_pad):
    n = i1.shape[0]

    @functools.partial(
        pl.kernel,
        mesh=_sc_mesh(),
        out_type=(
            jax.ShapeDtypeStruct((a_pad,), jnp.int32),
            jax.ShapeDtypeStruct((n,), jnp.int32),
            jax.ShapeDtypeStruct((n,), jnp.int32),
        ),
        scratch_types=[
            pltpu.VMEM((n,), jnp.int32), pltpu.VMEM((n,), jnp.int32),
            pltpu.VMEM((n,), jnp.int32), pltpu.VMEM((n,), jnp.int32),
            pltpu.VMEM((16,), jnp.int32),
            pltpu.VMEM((a_pad,), jnp.int32),
            pltpu.VMEM((n,), jnp.int32), pltpu.VMEM((n,), jnp.int32),
        ],
        compiler_params=_sc_compiler_params(),
    )
    def k(i1_hbm, i2_hbm, r1_hbm, r2_hbm, off_hbm,
          st_hbm, pos1_hbm, pos2_hbm,
          i1v, i2v, r1v, r2v, offv, stv, p1v, p2v):
        wid = lax.axis_index("s") * 2 + lax.axis_index("c")

        @pl.when(wid == 0)
        def _():
            pltpu.sync_copy(i1_hbm, i1v)
            pltpu.sync_copy(i2_hbm, i2v)
            pltpu.sync_copy(r1_hbm, r1v)
            pltpu.sync_copy(r2_hbm, r2v)
            pltpu.sync_copy(off_hbm, offv)
            @pl.loop(0, a_pad, step=16)
            def _(j):
                stv[pl.ds(j, 16)] = lax.rem(lax.iota(jnp.int32, 16) + j, n)

            @pl.loop(0, n, step=16)
            def _(t):
                sl = pl.ds(t, 16)
                toks = lax.iota(jnp.int32, 16) + t
                pos1 = plsc.load_gather(offv, [i1v[sl]]) + r1v[sl]
                p1v[sl] = pos1
                plsc.store_scatter(stv, [pos1], toks)
                pos2 = plsc.load_gather(offv, [i2v[sl]]) + r2v[sl]
                p2v[sl] = pos2
                plsc.store_scatter(stv, [pos2], toks)

            pltpu.sync_copy(stv, st_hbm)
            pltpu.sync_copy(p1v, pos1_hbm)
            pltpu.sync_copy(p2v, pos2_hbm)

    return k(i1, i2, r1, r2, off16)


def _gather_rows_call(table, idx, chunk, tag):
    nrows = idx.shape[0]
    d = table.shape[1]
    nw = 32
    per_w = nrows // nw
    nch = per_w // chunk

    def k(tab_hbm, idx_hbm, out_hbm, idxv, buf0, buf1,
          gs0, gs1, os0, os1):
        wid = lax.axis_index("s") * 2 + lax.axis_index("c")
        base = wid * per_w
        bufs, gsems, osems = (buf0, buf1), (gs0, gs1), (os0, os1)
        pltpu.sync_copy(idx_hbm.at[pl.ds(base, per_w)], idxv)

        def gather(c):
            return pltpu.make_async_copy(
                tab_hbm.at[idxv.at[pl.ds(c * chunk, chunk)]],
                bufs[c % 2], gsems[c % 2])

        def putout(c):
            return pltpu.make_async_copy(
                bufs[c % 2], out_hbm.at[pl.ds(base + c * chunk, chunk)],
                osems[c % 2])

        gather(0).start()
        for c in range(nch):
            if c + 1 < nch:
                if c >= 1:
                    putout(c - 1).wait()
                gather(c + 1).start()
            gather(c).wait()
            putout(c).start()
        if nch >= 2:
            putout(nch - 2).wait()
        putout(nch - 1).wait()

    k.__name__ = "gather_" + tag
    wrapped = pl.kernel(
        k,
        mesh=_sc_mesh(),
        out_type=jax.ShapeDtypeStruct((nrows, d), table.dtype),
        scratch_types=[
            pltpu.VMEM((per_w,), jnp.int32),
            pltpu.VMEM((chunk, d), table.dtype),
            pltpu.VMEM((chunk, d), table.dtype),
            pltpu.SemaphoreType.DMA, pltpu.SemaphoreType.DMA,
            pltpu.SemaphoreType.DMA, pltpu.SemaphoreType.DMA,
        ],
    )
    return wrapped(table, idx)


def _gemm_body(n_experts, te_ref, xs_ref, w1_ref, b1_ref, w2_ref, b2_ref,
               ys_ref):
    i = pl.program_id(0)

    @pl.when(te_ref[i] < n_experts)
    def _():
        h = jnp.dot(xs_ref[...], w1_ref[0],
                    preferred_element_type=jnp.float32)
        h = h + b1_ref[0]
        h = h * jax.nn.sigmoid(h)
        out = jnp.dot(h, w2_ref[0], preferred_element_type=jnp.float32)
        out = out + b2_ref[0]
        d2 = out.shape[1] // 2
        lo = lax.bitcast_convert_type(
            out[:, :d2].astype(jnp.bfloat16), jnp.uint16).astype(jnp.uint32)
        hi = lax.bitcast_convert_type(
            out[:, d2:].astype(jnp.bfloat16), jnp.uint16).astype(jnp.uint32)
        ys_ref[...] = lax.bitcast_convert_type(lo | (hi << 16), jnp.int32)


def _gemm_call(te, xs_bf, w1_bf, b1, w2_bf, b2):
    a_pad, d = xs_bf.shape
    f = w1_bf.shape[2]
    e = w1_bf.shape[0]
    nt = a_pad // T
    body = functools.partial(_gemm_body, e)
    grid_spec = pltpu.PrefetchScalarGridSpec(
        num_scalar_prefetch=1,
        grid=(nt,),
        in_specs=[
            pl.BlockSpec((T, d), lambda i, te: (i, 0)),
            pl.BlockSpec((1, d, f), lambda i, te: (te[i] % 8, 0, 0)),
            pl.BlockSpec((1, 1, f), lambda i, te: (te[i] % 8, 0, 0)),
            pl.BlockSpec((1, f, d), lambda i, te: (te[i] % 8, 0, 0)),
            pl.BlockSpec((1, 1, d), lambda i, te: (te[i] % 8, 0, 0)),
        ],
        out_specs=pl.BlockSpec((T, d // 2), lambda i, te: (i, 0)),
    )
    return pl.pallas_call(
        body,
        grid_spec=grid_spec,
        out_shape=jax.ShapeDtypeStruct((a_pad, d // 2), jnp.int32),
        compiler_params=pltpu.CompilerParams(
            dimension_semantics=("parallel",)),
    )(te, xs_bf, w1_bf, b1, w2_bf, b2)


def _unpack_bf16_pair(g):
    u = lax.bitcast_convert_type(g, jnp.uint32)
    lo = lax.bitcast_convert_type(
        (u & 0xFFFF).astype(jnp.uint16), jnp.bfloat16).astype(jnp.float32)
    hi = lax.bitcast_convert_type(
        (u >> 16).astype(jnp.uint16), jnp.bfloat16).astype(jnp.float32)
    return lo, hi


def _combine_body(g1_ref, g2_ref, p1_ref, p2_ref, o_ref):
    lo1, hi1 = _unpack_bf16_pair(g1_ref[...])
    lo2, hi2 = _unpack_bf16_pair(g2_ref[...])
    p1 = p1_ref[...]
    p2 = p2_ref[...]
    d2 = lo1.shape[1]
    o_ref[:, :d2] = p1 * lo1 + p2 * lo2
    o_ref[:, d2:] = p1 * hi1 + p2 * hi2


def _combine_call(g, p1, p2):
    n = p1.shape[0]
    d2 = g.shape[1]
    nblk = n // TB
    return pl.pallas_call(
        _combine_body,
        grid=(nblk,),
        in_specs=[
            pl.BlockSpec((TB, d2), lambda b: (b, 0)),
            pl.BlockSpec((TB, d2), lambda b: (b + nblk, 0)),
            pl.BlockSpec((TB, 1), lambda b: (b, 0)),
            pl.BlockSpec((TB, 1), lambda b: (b, 0)),
        ],
        out_specs=pl.BlockSpec((TB, 2 * d2), lambda b: (b, 0)),
        out_shape=jax.ShapeDtypeStruct((n, 2 * d2), jnp.float32),
        compiler_params=pltpu.CompilerParams(
            dimension_semantics=("parallel",)),
    )(g, g, p1, p2)


def kernel(x, gate_w, gate_b, w1, b1, w2, b2):
    x = jnp.asarray(x, jnp.float32)
    b, s, d = x.shape
    e = gate_w.shape[1]
    f = w1.shape[2]
    n = b * s
    a_pad = ((n * 2 + e * T) // T) * T

    x2 = x.reshape(n, d)
    gw_pad = jnp.pad(gate_w, ((0, 0), (0, LANES - e)))
    gb_pad = jnp.pad(gate_b, (0, LANES - e)).reshape(1, LANES)
    noise_pad = _noise_padded(b, s, e)

    i1, i2, r1, r2, p1, p2, off, te, auxv = _routing_call(
        x2, gw_pad, gb_pad, noise_pad, e)

    off16 = off[0, :16]
    st, pos1, pos2 = _positions_call(
        i1.reshape(n), i2.reshape(n), r1.reshape(n), r2.reshape(n),
        off16, a_pad)

    xs = _gather_rows_call(x2, st, chunk=32, tag="dispatch")

    ys = _gemm_call(te.reshape(LANES)[:a_pad // T], xs,
                    w1, b1.reshape(e, 1, f), w2, b2.reshape(e, 1, d))

    g = _gather_rows_call(ys, jnp.concatenate([pos1, pos2]), chunk=64,
                          tag="combine")
    out = _combine_call(g, p1, p2)

    final = out.reshape(b, s, d)
    topk = jnp.concatenate([i1, i2], axis=1).reshape(b, s, 2)
    aux = auxv[0, 0]
    return (final, topk, aux)


# speedup vs baseline: 1.2411x; 1.0381x over previous
import dataclasses
import functools

import jax
import jax.numpy as jnp
from jax import lax
from jax.experimental import pallas as pl
from jax.experimental.pallas import tpu as pltpu
from jax.experimental.pallas import tpu_sc as plsc

LANES = 128
TB = 512
T = 256
NEG = -1000000000.0


def _noise_padded(b, s, e):
    noise = jax.random.normal(jax.random.key(42), (b, s, e),
                              dtype=jnp.float32) * 0.01
    return jnp.pad(noise.reshape(b * s, e), ((0, 0), (0, LANES - e)),
                   constant_values=NEG)


def _routing_body(n_tokens, n_experts, x_ref, gw_ref, gb_ref, noise_ref,
                  i1_ref, i2_ref, r1_ref, r2_ref, p1_ref, p2_ref,
                  off_ref, te_ref, aux_ref, carry, usage):
    blk = pl.program_id(0)
    nblk = pl.num_programs(0)

    @pl.when(blk == 0)
    def _():
        carry[...] = jnp.zeros_like(carry)
        usage[...] = jnp.zeros_like(usage)

    logits = jnp.dot(x_ref[...].astype(jnp.bfloat16),
                     gw_ref[...].astype(jnp.bfloat16),
                     preferred_element_type=jnp.float32)
    logits = logits + gb_ref[...] + noise_ref[...]

    lane = lax.broadcasted_iota(jnp.int32, (TB, LANES), 1)
    m1 = jnp.max(logits, axis=1, keepdims=True)
    i1 = jnp.min(jnp.where(logits == m1, lane, LANES), axis=1, keepdims=True)
    oh1 = lane == i1
    l2 = jnp.where(oh1, NEG, logits)
    m2 = jnp.max(l2, axis=1, keepdims=True)
    i2 = jnp.min(jnp.where(l2 == m2, lane, LANES), axis=1, keepdims=True)
    oh2 = lane == i2

    masked = jnp.where(logits >= m2, logits, NEG)
    ex = jnp.exp(masked - m1)
    probs = ex / jnp.sum(ex, axis=1, keepdims=True)
    p1 = jnp.sum(jnp.where(oh1, probs, 0.0), axis=1, keepdims=True)
    p2 = jnp.sum(jnp.where(oh2, probs, 0.0), axis=1, keepdims=True)
    usage[...] += jnp.sum(probs, axis=0, keepdims=True)

    ohb = (oh1 | oh2).astype(jnp.float32)
    row = lax.broadcasted_iota(jnp.int32, (TB, TB), 0)
    col = lax.broadcasted_iota(jnp.int32, (TB, TB), 1)
    tri = (row > col).astype(jnp.float32)
    pre = jnp.dot(tri, ohb, preferred_element_type=jnp.float32)
    base = pre + carry[...]
    r1 = jnp.sum(jnp.where(oh1, base, 0.0), axis=1, keepdims=True)
    r2 = jnp.sum(jnp.where(oh2, base, 0.0), axis=1, keepdims=True)
    carry[...] += jnp.sum(ohb, axis=0, keepdims=True)

    i1_ref[...] = i1.astype(jnp.int32)
    i2_ref[...] = i2.astype(jnp.int32)
    r1_ref[...] = r1.astype(jnp.int32)
    r2_ref[...] = r2.astype(jnp.int32)
    p1_ref[...] = p1
    p2_ref[...] = p2

    @pl.when(blk == nblk - 1)
    def _():
        counts = carry[...]
        padded = jnp.ceil(counts / T) * T
        r128 = lax.broadcasted_iota(jnp.int32, (LANES, LANES), 0)
        c128 = lax.broadcasted_iota(jnp.int32, (LANES, LANES), 1)
        ut = (r128 < c128).astype(jnp.float32)
        pad8 = jnp.broadcast_to(padded, (8, LANES))
        off_excl = jnp.dot(pad8, ut, preferred_element_type=jnp.float32)[0:1]
        off_incl = off_excl + padded
        off_ref[...] = off_excl.astype(jnp.int32)

        tstart = lax.broadcasted_iota(jnp.int32, (LANES, 1), 0) * T
        offi = jnp.broadcast_to(off_incl, (LANES, LANES)).astype(jnp.int32)
        m = jnp.where(c128 < n_experts, (tstart >= offi).astype(jnp.int32), 0)
        te = jnp.sum(m, axis=1, keepdims=True)
        te_ref[...] = jnp.where(te >= n_experts, n_experts - 1 + 8, te)

        tot = jnp.sum(usage[...])
        imp = usage[...] / tot
        mean = jnp.sum(imp) / n_experts
        lr = lax.broadcasted_iota(jnp.int32, (1, LANES), 1)
        diff = jnp.where(lr < n_experts, imp - mean, 0.0)
        var = jnp.sum(diff * diff) / n_experts
        aux = jnp.sqrt(var) / (mean + 1e-10)
        aux_ref[...] = jnp.full((1, LANES), aux, jnp.float32)


def _routing_call(x2, gw_pad, gb_pad, noise_pad, n_experts):
    n = x2.shape[0]
    d = x2.shape[1]
    nblk = n // TB
    body = functools.partial(_routing_body, n, n_experts)
    outs = (
        jax.ShapeDtypeStruct((n, 1), jnp.int32),
        jax.ShapeDtypeStruct((n, 1), jnp.int32),
        jax.ShapeDtypeStruct((n, 1), jnp.int32),
        jax.ShapeDtypeStruct((n, 1), jnp.int32),
        jax.ShapeDtypeStruct((n, 1), jnp.float32),
        jax.ShapeDtypeStruct((n, 1), jnp.float32),
        jax.ShapeDtypeStruct((1, LANES), jnp.int32),
        jax.ShapeDtypeStruct((LANES, 1), jnp.int32),
        jax.ShapeDtypeStruct((1, LANES), jnp.float32),
    )
    tok = lambda b: (b, 0)
    fixed = lambda b: (0, 0)
    return pl.pallas_call(
        body,
        grid=(nblk,),
        in_specs=[
            pl.BlockSpec((TB, d), tok),
            pl.BlockSpec((d, LANES), fixed),
            pl.BlockSpec((1, LANES), fixed),
            pl.BlockSpec((TB, LANES), tok),
        ],
        out_specs=(
            pl.BlockSpec((TB, 1), tok), pl.BlockSpec((TB, 1), tok),
            pl.BlockSpec((TB, 1), tok), pl.BlockSpec((TB, 1), tok),
            pl.BlockSpec((TB, 1), tok), pl.BlockSpec((TB, 1), tok),
            pl.BlockSpec((1, LANES), fixed),
            pl.BlockSpec((LANES, 1), fixed),
            pl.BlockSpec((1, LANES), fixed),
        ),
        out_shape=outs,
        scratch_shapes=[
            pltpu.VMEM((1, LANES), jnp.float32),
            pltpu.VMEM((1, LANES), jnp.float32),
        ],
        compiler_params=pltpu.CompilerParams(
            dimension_semantics=("arbitrary",)),
    )(x2, gw_pad, gb_pad, noise_pad)


def _sc_mesh():
    return plsc.VectorSubcoreMesh(core_axis_name="c", subcore_axis_name="s")


def _sc_compiler_params():
    cp = pltpu.CompilerParams()
    if "needs_layout_passes" in pltpu.CompilerParams.__dataclass_fields__:
        cp = dataclasses.replace(cp, needs_layout_passes=False)
    return cp


def _positions_call(i1, i2, r1, r2, off16, a_pad):
    n = i1.shape[0]

    @functools.partial(
        pl.kernel,
        mesh=_sc_mesh(),
        out_type=(
            jax.ShapeDtypeStruct((a_pad,), jnp.int32),
            jax.ShapeDtypeStruct((n,), jnp.int32),
            jax.ShapeDtypeStruct((n,), jnp.int32),
        ),
        scratch_types=[
            pltpu.VMEM((n,), jnp.int32), pltpu.VMEM((n,), jnp.int32),
            pltpu.VMEM((n,), jnp.int32), pltpu.VMEM((n,), jnp.int32),
            pltpu.VMEM((16,), jnp.int32),
            pltpu.VMEM((a_pad,), jnp.int32),
            pltpu.VMEM((n,), jnp.int32), pltpu.VMEM((n,), jnp.int32),
        ],
        compiler_params=_sc_compiler_params(),
    )
    def k(i1_hbm, i2_hbm, r1_hbm, r2_hbm, off_hbm,
          st_hbm, pos1_hbm, pos2_hbm,
          i1v, i2v, r1v, r2v, offv, stv, p1v, p2v):
        wid = lax.axis_index("s") * 2 + lax.axis_index("c")

        @pl.when(wid == 0)
        def _():
            pltpu.sync_copy(i1_hbm, i1v)
            pltpu.sync_copy(i2_hbm, i2v)
            pltpu.sync_copy(r1_hbm, r1v)
            pltpu.sync_copy(r2_hbm, r2v)
            pltpu.sync_copy(off_hbm, offv)
            @pl.loop(0, a_pad, step=16)
            def _(j):
                stv[pl.ds(j, 16)] = lax.rem(lax.iota(jnp.int32, 16) + j, n)

            @pl.loop(0, n, step=16)
            def _(t):
                sl = pl.ds(t, 16)
                toks = lax.iota(jnp.int32, 16) + t
                pos1 = plsc.load_gather(offv, [i1v[sl]]) + r1v[sl]
                p1v[sl] = pos1
                plsc.store_scatter(stv, [pos1], toks)
                pos2 = plsc.load_gather(offv, [i2v[sl]]) + r2v[sl]
                p2v[sl] = pos2
                plsc.store_scatter(stv, [pos2], toks)

            pltpu.sync_copy(stv, st_hbm)
            pltpu.sync_copy(p1v, pos1_hbm)
            pltpu.sync_copy(p2v, pos2_hbm)

    return k(i1, i2, r1, r2, off16)


def _gather_rows_call(table, idx, chunk, tag):
    nrows = idx.shape[0]
    d = table.shape[1]
    nw = 32
    per_w = nrows // nw
    nch = per_w // chunk

    def k(tab_hbm, idx_hbm, out_hbm, idxv, buf0, buf1,
          gs0, gs1, os0, os1):
        wid = lax.axis_index("s") * 2 + lax.axis_index("c")
        base = wid * per_w
        bufs, gsems, osems = (buf0, buf1), (gs0, gs1), (os0, os1)
        pltpu.sync_copy(idx_hbm.at[pl.ds(base, per_w)], idxv)

        def gather(c):
            return pltpu.make_async_copy(
                tab_hbm.at[idxv.at[pl.ds(c * chunk, chunk)]],
                bufs[c % 2], gsems[c % 2])

        def putout(c):
            return pltpu.make_async_copy(
                bufs[c % 2], out_hbm.at[pl.ds(base + c * chunk, chunk)],
                osems[c % 2])

        gather(0).start()
        for c in range(nch):
            if c + 1 < nch:
                if c >= 1:
                    putout(c - 1).wait()
                gather(c + 1).start()
            gather(c).wait()
            putout(c).start()
        if nch >= 2:
            putout(nch - 2).wait()
        putout(nch - 1).wait()

    k.__name__ = "gather_" + tag
    wrapped = pl.kernel(
        k,
        mesh=_sc_mesh(),
        out_type=jax.ShapeDtypeStruct((nrows, d), table.dtype),
        scratch_types=[
            pltpu.VMEM((per_w,), jnp.int32),
            pltpu.VMEM((chunk, d), table.dtype),
            pltpu.VMEM((chunk, d), table.dtype),
            pltpu.SemaphoreType.DMA, pltpu.SemaphoreType.DMA,
            pltpu.SemaphoreType.DMA, pltpu.SemaphoreType.DMA,
        ],
    )
    return wrapped(table, idx)


def _gemm_body(n_experts, te_ref, xs_ref, w1_ref, b1_ref, w2_ref, b2_ref,
               ys_ref):
    i = pl.program_id(0)

    @pl.when(te_ref[i] < n_experts)
    def _():
        h = jnp.dot(xs_ref[...], w1_ref[0],
                    preferred_element_type=jnp.float32)
        h = h + b1_ref[0]
        h = h * jax.nn.sigmoid(h)
        out = jnp.dot(h, w2_ref[0], preferred_element_type=jnp.float32)
        out = out + b2_ref[0]
        d2 = out.shape[1] // 2
        lo = lax.bitcast_convert_type(
            out[:, :d2].astype(jnp.bfloat16), jnp.uint16).astype(jnp.uint32)
        hi = lax.bitcast_convert_type(
            out[:, d2:].astype(jnp.bfloat16), jnp.uint16).astype(jnp.uint32)
        ys_ref[...] = lax.bitcast_convert_type(lo | (hi << 16), jnp.int32)


def _gemm_call(te, xs_bf, w1_bf, b1, w2_bf, b2):
    a_pad, d = xs_bf.shape
    f = w1_bf.shape[2]
    e = w1_bf.shape[0]
    nt = a_pad // T
    body = functools.partial(_gemm_body, e)
    grid_spec = pltpu.PrefetchScalarGridSpec(
        num_scalar_prefetch=1,
        grid=(nt,),
        in_specs=[
            pl.BlockSpec((T, d), lambda i, te: (i, 0)),
            pl.BlockSpec((1, d, f), lambda i, te: (te[i] % 8, 0, 0)),
            pl.BlockSpec((1, 1, f), lambda i, te: (te[i] % 8, 0, 0)),
            pl.BlockSpec((1, f, d), lambda i, te: (te[i] % 8, 0, 0)),
            pl.BlockSpec((1, 1, d), lambda i, te: (te[i] % 8, 0, 0)),
        ],
        out_specs=pl.BlockSpec((T, d // 2), lambda i, te: (i, 0)),
    )
    return pl.pallas_call(
        body,
        grid_spec=grid_spec,
        out_shape=jax.ShapeDtypeStruct((a_pad, d // 2), jnp.int32),
        compiler_params=pltpu.CompilerParams(
            dimension_semantics=("parallel",)),
    )(te, xs_bf, w1_bf, b1, w2_bf, b2)


def _unpack_bf16_pair(g):
    u = lax.bitcast_convert_type(g, jnp.uint32)
    lo = lax.bitcast_convert_type(
        (u & 0xFFFF).astype(jnp.uint16), jnp.bfloat16).astype(jnp.float32)
    hi = lax.bitcast_convert_type(
        (u >> 16).astype(jnp.uint16), jnp.bfloat16).astype(jnp.float32)
    return lo, hi


def _combine_body(g1_ref, g2_ref, p1_ref, p2_ref, o_ref):
    lo1, hi1 = _unpack_bf16_pair(g1_ref[...])
    lo2, hi2 = _unpack_bf16_pair(g2_ref[...])
    p1 = p1_ref[...]
    p2 = p2_ref[...]
    d2 = lo1.shape[1]
    o_ref[:, :d2] = p1 * lo1 + p2 * lo2
    o_ref[:, d2:] = p1 * hi1 + p2 * hi2


def _combine_call(g, p1, p2):
    n = p1.shape[0]
    d2 = g.shape[1]
    nblk = n // TB
    return pl.pallas_call(
        _combine_body,
        grid=(nblk,),
        in_specs=[
            pl.BlockSpec((TB, d2), lambda b: (b, 0)),
            pl.BlockSpec((TB, d2), lambda b: (b + nblk, 0)),
            pl.BlockSpec((TB, 1), lambda b: (b, 0)),
            pl.BlockSpec((TB, 1), lambda b: (b, 0)),
        ],
        out_specs=pl.BlockSpec((TB, 2 * d2), lambda b: (b, 0)),
        out_shape=jax.ShapeDtypeStruct((n, 2 * d2), jnp.float32),
        compiler_params=pltpu.CompilerParams(
            dimension_semantics=("parallel",)),
    )(g, g, p1, p2)


def kernel(x, gate_w, gate_b, w1, b1, w2, b2):
    x = jnp.asarray(x, jnp.float32)
    b, s, d = x.shape
    e = gate_w.shape[1]
    f = w1.shape[2]
    n = b * s
    a_pad = ((n * 2 + e * T) // T) * T

    x2 = x.reshape(n, d)
    gw_pad = jnp.pad(gate_w, ((0, 0), (0, LANES - e)))
    gb_pad = jnp.pad(gate_b, (0, LANES - e)).reshape(1, LANES)
    noise_pad = _noise_padded(b, s, e)

    i1, i2, r1, r2, p1, p2, off, te, auxv = _routing_call(
        x2, gw_pad, gb_pad, noise_pad, e)

    off16 = off[0, :16]
    st, pos1, pos2 = _positions_call(
        i1.reshape(n), i2.reshape(n), r1.reshape(n), r2.reshape(n),
        off16, a_pad)

    xs = _gather_rows_call(x2, st, chunk=40, tag="dispatch")

    ys = _gemm_call(te.reshape(LANES)[:a_pad // T], xs,
                    w1, b1.reshape(e, 1, f), w2, b2.reshape(e, 1, d))

    g = _gather_rows_call(ys, jnp.concatenate([pos1, pos2]), chunk=64,
                          tag="combine")
    out = _combine_call(g, p1, p2)

    final = out.reshape(b, s, d)
    topk = jnp.concatenate([i1, i2], axis=1).reshape(b, s, 2)
    aux = auxv[0, 0]
    return (final, topk, aux)
